# Initial kernel scaffold; baseline (speedup 1.0000x reference)
import jax
import jax.numpy as jnp
from jax.experimental import pallas as pl


def kernel(tensor, nodes_per_img, weight, bias, mean_scale):
    raise NotImplementedError("write your pallas kernel here")


# speedup vs baseline: 8.2568x; 8.2568x over previous
import functools

import jax
import jax.numpy as jnp
from jax.experimental import pallas as pl
from jax.experimental.pallas import tpu as pltpu


def _stats_body(x_ref, lo_ref, hi_ref, c_ref, invc_ref, ms_ref, w_ref, b_ref,
                a_out, c_out, sum_s, sq_s, *, R, N, G):
    i = pl.program_id(0)

    @pl.when(i == 0)
    def _():
        sum_s[...] = jnp.zeros_like(sum_s)
        sq_s[...] = jnp.zeros_like(sq_s)

    rg_row = i * R + jax.lax.broadcasted_iota(jnp.int32, (1, R), 1)
    oh = ((rg_row >= lo_ref[...]) & (rg_row < hi_ref[...])).astype(jnp.float32)
    rg_col = i * R + jax.lax.broadcasted_iota(jnp.int32, (R, 1), 0)
    xm = jnp.where(rg_col < N, x_ref[...], 0.0)
    sum_s[...] += jnp.dot(oh, xm, preferred_element_type=jnp.float32)
    sq_s[...] += jnp.dot(oh, xm * xm, preferred_element_type=jnp.float32)

    @pl.when(i == G - 1)
    def _():
        s = sum_s[...]
        mean = s * invc_ref[...]
        msm = mean * ms_ref[...]
        varsum = sq_s[...] - 2.0 * msm * s + c_ref[...] * msm * msm
        a = w_ref[...] * jax.lax.rsqrt(varsum * invc_ref[...] + 1e-6)
        a_out[...] = a
        c_out[...] = b_ref[...] - a * msm


def _apply_body(x_ref, a_ref, c_ref, lo_ref, hi_ref, o_ref, *, R):
    i = pl.program_id(0)
    rg_col = i * R + jax.lax.broadcasted_iota(jnp.int32, (R, 1), 0)
    oh = ((rg_col >= lo_ref[...]) & (rg_col < hi_ref[...])).astype(jnp.float32)
    ar = jnp.dot(oh, a_ref[...], preferred_element_type=jnp.float32)
    cr = jnp.dot(oh, c_ref[...], preferred_element_type=jnp.float32)
    o_ref[...] = ar * x_ref[...] + cr


@jax.jit
def kernel(tensor, nodes_per_img, weight, bias, mean_scale):
    N, D = tensor.shape
    B = nodes_per_img.shape[0]
    R = 512
    G = pl.cdiv(N, R)
    Bp = 384

    counts = nodes_per_img.astype(jnp.float32)
    sizes = nodes_per_img.astype(jnp.int32)
    hi = jnp.cumsum(sizes)
    lo = hi - sizes
    lo_p = jnp.full((Bp,), N, jnp.int32).at[:B].set(lo)
    hi_p = jnp.full((Bp,), N, jnp.int32).at[:B].set(hi)
    c_col = jnp.zeros((Bp, 1), jnp.float32).at[:B, 0].set(counts)
    invc_col = 1.0 / (c_col + 1e-6)

    def const(shape):
        return pl.BlockSpec(shape, lambda i: (0, 0))

    a_mat, c_mat = pl.pallas_call(
        functools.partial(_stats_body, R=R, N=N, G=G),
        grid=(G,),
        in_specs=[
            pl.BlockSpec((R, D), lambda i: (i, 0)),
            const((Bp, 1)), const((Bp, 1)), const((Bp, 1)), const((Bp, 1)),
            const((1, D)), const((1, D)), const((1, D)),
        ],
        out_specs=[const((Bp, D)), const((Bp, D))],
        out_shape=[
            jax.ShapeDtypeStruct((Bp, D), jnp.float32),
            jax.ShapeDtypeStruct((Bp, D), jnp.float32),
        ],
        scratch_shapes=[
            pltpu.VMEM((Bp, D), jnp.float32),
            pltpu.VMEM((Bp, D), jnp.float32),
        ],
    )(
        tensor,
        lo_p.reshape(Bp, 1), hi_p.reshape(Bp, 1),
        c_col, invc_col,
        mean_scale.reshape(1, D), weight.reshape(1, D), bias.reshape(1, D),
    )

    out = pl.pallas_call(
        functools.partial(_apply_body, R=R),
        grid=(G,),
        in_specs=[
            pl.BlockSpec((R, D), lambda i: (i, 0)),
            const((Bp, D)), const((Bp, D)), const((1, Bp)), const((1, Bp)),
        ],
        out_specs=pl.BlockSpec((R, D), lambda i: (i, 0)),
        out_shape=jax.ShapeDtypeStruct((N, D), jnp.float32),
    )(tensor, a_mat, c_mat, lo_p.reshape(1, Bp), hi_p.reshape(1, Bp))
    return out
